# Initial kernel scaffold; baseline (speedup 1.0000x reference)
import jax
import jax.numpy as jnp
from jax.experimental import pallas as pl


def kernel(user_weight, item_weight, train_user_ids, train_item_ids):
    raise NotImplementedError("write your pallas kernel here")


# speedup vs baseline: 5.2488x; 5.2488x over previous
import functools

import jax
import jax.numpy as jnp
from jax import lax
from jax.experimental import pallas as pl
from jax.experimental.pallas import tpu as pltpu
from jax.experimental.pallas import tpu_sc as plsc

N_USERS = 50000
N_ITEMS = 50000
EMB_DIM = 64
N_LAYERS = 3

NPAD = 50176
NP = 2 * NPAD
D = EMB_DIM
W = NPAD // 2
RPT = W // 16
ACC_ROWS = W + 8
DUMP = W
CH = 384
NSUB = CH // 128
NCHUNK = 100
EPT = CH * NCHUNK
EPAD = EPT * 16
BIG = 1 << 20


def _propagate_body(f_hbm, uids_hbm, iids_hbm, out_hbm,
                    uid_v, iid_v, srcx, dstx, rows_v, acc):
    cid = lax.axis_index("c")
    sid = lax.axis_index("s")
    ebase = sid * EPT
    wbase = cid * W

    for p in range(2):
        obase = p * NPAD + wbase

        def _zb(i, _):
            zero16 = jnp.zeros((16,), jnp.float32)
            for q in range(4):
                rows_v[i, pl.ds(q * 16, 16)] = zero16
            return 0
        lax.fori_loop(0, CH, _zb, 0)
        for z in range(4):
            pltpu.sync_copy(rows_v, acc.at[pl.ds(sid * RPT + z * CH, CH)])
        pltpu.sync_copy(rows_v.at[pl.ds(0, 32)],
                        acc.at[pl.ds(sid * RPT + 4 * CH, 32)])
        plsc.subcore_barrier()

        def _chunk(k, _):
            eo = ebase + k * CH
            pltpu.sync_copy(uids_hbm.at[pl.ds(eo, CH)], uid_v)
            pltpu.sync_copy(iids_hbm.at[pl.ds(eo, CH)], iid_v)

            def _ix(m, _):
                u = uid_v[pl.ds(m * 16, 16)]
                i = iid_v[pl.ds(m * 16, 16)]
                if p == 0:
                    dst = u
                    src = i + NPAD
                else:
                    dst = i
                    src = u
                dl = dst - wbase
                oob = (dl < 0) | (dl >= W)
                dl = jnp.where(oob, DUMP, dl)
                src = jnp.minimum(src, NP - 1)
                row = m // 8
                col = (m % 8) * 16
                srcx[row, pl.ds(col, 16)] = src
                dstx[row, pl.ds(col, 16)] = dl
                return 0
            lax.fori_loop(0, CH // 16, _ix, 0)

            for j in range(NSUB):
                pltpu.sync_copy(f_hbm.at[srcx.at[j]],
                                rows_v.at[pl.ds(j * 128, 128)])
            for j in range(NSUB):
                pltpu.sync_copy(rows_v.at[pl.ds(j * 128, 128)],
                                acc.at[dstx.at[j]], add=True)
            return 0
        lax.fori_loop(0, NCHUNK, _chunk, 0)

        plsc.subcore_barrier()
        rb = sid * RPT
        pltpu.sync_copy(acc.at[pl.ds(rb, RPT)],
                        out_hbm.at[pl.ds(obase + rb, RPT)])
        plsc.subcore_barrier()


_propagate = functools.partial(
    pl.kernel,
    out_type=jax.ShapeDtypeStruct((NP, D), jnp.float32),
    mesh=plsc.VectorSubcoreMesh(core_axis_name="c", subcore_axis_name="s"),
    compiler_params=pltpu.CompilerParams(use_tc_tiling_on_sc=False),
    scratch_types=[
        pltpu.VMEM((CH,), jnp.int32),
        pltpu.VMEM((CH,), jnp.int32),
        pltpu.VMEM((NSUB, 128), jnp.int32),
        pltpu.VMEM((NSUB, 128), jnp.int32),
        pltpu.VMEM((CH, D), jnp.float32),
        pltpu.VMEM_SHARED((ACC_ROWS, D), jnp.float32),
    ],
)(_propagate_body)


def kernel(user_weight, item_weight, train_user_ids, train_item_ids):
    emb0 = jnp.zeros((NP, D), jnp.float32)
    emb0 = emb0.at[:N_USERS].set(user_weight)
    emb0 = emb0.at[NPAD:NPAD + N_ITEMS].set(item_weight)

    deg_u = jnp.bincount(train_user_ids, length=N_USERS).astype(jnp.float32)
    deg_i = jnp.bincount(train_item_ids, length=N_ITEMS).astype(jnp.float32)
    deg = jnp.zeros((NP,), jnp.float32)
    deg = deg.at[:N_USERS].set(deg_u)
    deg = deg.at[NPAD:NPAD + N_ITEMS].set(deg_i)
    dinvs = jnp.where(deg > 0, lax.rsqrt(jnp.where(deg > 0, deg, 1.0)), 0.0)
    dinv = jnp.where(deg > 0, 1.0 / jnp.where(deg > 0, deg, 1.0), 0.0)

    npad_e = EPAD - train_user_ids.shape[0]
    pad_ids = jnp.full((npad_e,), BIG, jnp.int32)
    uids = jnp.concatenate([train_user_ids.astype(jnp.int32), pad_ids])
    iids = jnp.concatenate([train_item_ids.astype(jnp.int32), pad_ids])

    f = dinvs[:, None] * emb0
    total = emb0
    for layer in range(N_LAYERS):
        s = _propagate(f, uids, iids)
        total = total + dinvs[:, None] * s
        if layer < N_LAYERS - 1:
            f = dinv[:, None] * s

    final = total * (1.0 / (N_LAYERS + 1))
    return (final[:N_USERS], final[NPAD:NPAD + N_ITEMS])


# speedup vs baseline: 5.7817x; 1.1015x over previous
import functools

import jax
import jax.numpy as jnp
from jax import lax
from jax.experimental import pallas as pl
from jax.experimental.pallas import tpu as pltpu
from jax.experimental.pallas import tpu_sc as plsc

N_USERS = 50000
N_ITEMS = 50000
EMB_DIM = 64
N_LAYERS = 3

NPAD = 50176
NP = 2 * NPAD
D = EMB_DIM
W = NPAD // 2
RPT = W // 16
ACC_ROWS = W + 8
DUMP = W
CH = 128
EPT = CH * 300
EPAD = EPT * 16
BIG = 1 << 20


def _propagate_body(f_hbm, sidx_hbm, didx_hbm, out_hbm,
                    sbuf, dbuf, rows, acc, isem, gsem, ssem):
    cid = lax.axis_index("c")
    sid = lax.axis_index("s")
    ebase = sid * EPT
    NCH = EPT // CH

    for p in range(2):
        obase = p * NPAD + cid * W
        soff = p * EPAD + ebase
        doff = (p * 2 + cid) * EPAD + ebase

        def _zb(i, _):
            zero16 = jnp.zeros((16,), jnp.float32)
            for q in range(4):
                rows[0, i, pl.ds(q * 16, 16)] = zero16
            return 0
        lax.fori_loop(0, CH, _zb, 0)
        for z in range(12):
            pltpu.sync_copy(rows.at[0], acc.at[pl.ds(sid * RPT + z * CH, CH)])
        pltpu.sync_copy(rows.at[0, pl.ds(0, 32)],
                        acc.at[pl.ds(sid * RPT + 12 * CH, 32)])
        plsc.subcore_barrier()

        def fire_idx(k):
            pltpu.async_copy(sidx_hbm.at[pl.ds(soff + k * CH, CH)],
                             sbuf.at[k % 2], isem)
            pltpu.async_copy(didx_hbm.at[pl.ds(doff + k * CH, CH)],
                             dbuf.at[k % 3], isem)

        def wait_idx(k):
            pltpu.make_async_copy(sidx_hbm.at[pl.ds(soff + k * CH, CH)],
                                  sbuf.at[k % 2], isem).wait()
            pltpu.make_async_copy(didx_hbm.at[pl.ds(doff + k * CH, CH)],
                                  dbuf.at[k % 3], isem).wait()

        def fire_gather(k):
            pltpu.async_copy(f_hbm.at[sbuf.at[k % 2]], rows.at[k % 2], gsem)

        def wait_gather(k):
            pltpu.make_async_copy(f_hbm.at[sbuf.at[k % 2]],
                                  rows.at[k % 2], gsem).wait()

        def fire_scatter(k):
            pltpu.async_copy(rows.at[k % 2], acc.at[dbuf.at[k % 3]],
                             ssem, add=True)

        def wait_scatter(k):
            pltpu.make_async_copy(rows.at[k % 2],
                                  acc.at[dbuf.at[k % 3]], ssem).wait()

        fire_idx(0)
        fire_idx(1)
        wait_idx(0)
        fire_gather(0)

        def _chunk(k, _):
            wait_gather(k)
            fire_scatter(k)

            @pl.when(k >= 1)
            def _():
                wait_scatter(k - 1)

            @pl.when(k + 1 < NCH)
            def _():
                wait_idx(k + 1)
                fire_gather(k + 1)

            @pl.when(k + 2 < NCH)
            def _():
                fire_idx(k + 2)
            return 0
        lax.fori_loop(0, NCH, _chunk, 0)
        wait_scatter(NCH - 1)

        plsc.subcore_barrier()
        rb = sid * RPT
        pltpu.sync_copy(acc.at[pl.ds(rb, RPT)],
                        out_hbm.at[pl.ds(obase + rb, RPT)])
        plsc.subcore_barrier()


_propagate = functools.partial(
    pl.kernel,
    out_type=jax.ShapeDtypeStruct((NP, D), jnp.float32),
    mesh=plsc.VectorSubcoreMesh(core_axis_name="c", subcore_axis_name="s"),
    compiler_params=pltpu.CompilerParams(use_tc_tiling_on_sc=False),
    scratch_types=[
        pltpu.VMEM((2, 128), jnp.int32),
        pltpu.VMEM((3, 128), jnp.int32),
        pltpu.VMEM((2, CH, D), jnp.float32),
        pltpu.VMEM_SHARED((ACC_ROWS, D), jnp.float32),
        pltpu.SemaphoreType.DMA,
        pltpu.SemaphoreType.DMA,
        pltpu.SemaphoreType.DMA,
    ],
)(_propagate_body)


def kernel(user_weight, item_weight, train_user_ids, train_item_ids):
    emb0 = jnp.zeros((NP, D), jnp.float32)
    emb0 = emb0.at[:N_USERS].set(user_weight)
    emb0 = emb0.at[NPAD:NPAD + N_ITEMS].set(item_weight)

    deg_u = jnp.bincount(train_user_ids, length=N_USERS).astype(jnp.float32)
    deg_i = jnp.bincount(train_item_ids, length=N_ITEMS).astype(jnp.float32)
    deg = jnp.zeros((NP,), jnp.float32)
    deg = deg.at[:N_USERS].set(deg_u)
    deg = deg.at[NPAD:NPAD + N_ITEMS].set(deg_i)
    dinvs = jnp.where(deg > 0, lax.rsqrt(jnp.where(deg > 0, deg, 1.0)), 0.0)
    dinv = jnp.where(deg > 0, 1.0 / jnp.where(deg > 0, deg, 1.0), 0.0)

    npad_e = EPAD - train_user_ids.shape[0]
    pad_ids = jnp.full((npad_e,), BIG, jnp.int32)
    uids = jnp.concatenate([train_user_ids.astype(jnp.int32), pad_ids])
    iids = jnp.concatenate([train_item_ids.astype(jnp.int32), pad_ids])

    src0 = jnp.minimum(iids + NPAD, NP - 1)
    src1 = jnp.minimum(uids, NP - 1)
    sidx = jnp.concatenate([src0, src1])

    def _dloc(ids, c):
        dl = ids - c * W
        return jnp.where((dl < 0) | (dl >= W), DUMP, dl)
    didx = jnp.concatenate([_dloc(uids, 0), _dloc(uids, 1),
                            _dloc(iids, 0), _dloc(iids, 1)])

    f = dinvs[:, None] * emb0
    total = emb0
    for layer in range(N_LAYERS):
        s = _propagate(f, sidx, didx)
        total = total + dinvs[:, None] * s
        if layer < N_LAYERS - 1:
            f = dinv[:, None] * s

    final = total * (1.0 / (N_LAYERS + 1))
    return (final[:N_USERS], final[NPAD:NPAD + N_ITEMS])
